# Initial kernel scaffold; baseline (speedup 1.0000x reference)
import jax
import jax.numpy as jnp
from jax.experimental import pallas as pl


def kernel(x, max_buf, p99_99_buf, p99_9_buf, p99_buf):
    raise NotImplementedError("write your pallas kernel here")


# speedup vs baseline: 80.9521x; 80.9521x over previous
import functools

import jax
import jax.numpy as jnp
from jax import lax
from jax.experimental import pallas as pl
from jax.experimental.pallas import tpu as pltpu
from jax.experimental.pallas import tpu_sc as plsc

_GAMMA = 0.99
_NC, _NS, _L = 2, 16, 16
_NW = _NC * _NS
_CHUNK = 8192

_B1, _B2, _B3 = 512, 2048, 2048
_U1 = 8
_H1 = _B1 * _U1 * _L
_H23 = _B2 * _L

_MESH = dict(core_axis_name="c", subcore_axis_name="s",
             num_cores=_NC, num_subcores=_NS)


def _wid():
    return lax.axis_index("s") * _NC + lax.axis_index("c")


def _zero_hist(hist_v, nwords):
    zeros = jnp.zeros((_L,), jnp.int32)
    @plsc.parallel_loop(0, nwords // _L, unroll=8)
    def _(i):
        hist_v[pl.ds(i * _L, _L)] = zeros


def _p1_body(n, x_hbm, hist_hbm, max_hbm, xo_hbm,
             b0, b1, b2, b3, hist_v, max_v,
             si0, si1, si2, si3, so0, so1, so2, so3):
    per = n // _NW
    base = _wid() * per
    nchunk = per // _CHUNK
    bufs = (b0, b1, b2, b3)
    sin = (si0, si1, si2, si3)
    sout = (so0, so1, so2, so3)
    _zero_hist(hist_v, _H1)
    lane = lax.iota(jnp.int32, _L)
    ones = jnp.ones((_L,), jnp.int32)

    def start_in(ci, buf, sem):
        cc = jnp.minimum(ci, nchunk - 1)
        pltpu.async_copy(x_hbm.at[pl.ds(base + cc * _CHUNK, _CHUNK)], buf, sem)

    def compute(buf, mx):
        def body(j, mx):
            u = plsc.bitcast(buf[pl.ds(j * _L, _L)], jnp.int32)
            a = u & jnp.int32(0x7FFFFFFF)
            mx = jnp.maximum(mx, a)
            co = ((j.astype(jnp.int32) & 7) << 4) | lane
            idx = ((a >> 15) & jnp.int32(0xFF80)) | co
            plsc.addupdate_scatter(hist_v, [idx], ones)
            return mx
        return plsc.parallel_loop(0, _CHUNK // _L, carry=mx, unroll=16)(body)

    for b in range(4):
        start_in(jnp.int32(b), bufs[b], sin[b])

    def outer(i4, mx):
        c0 = 4 * i4
        for b in range(4):
            c = c0 + b
            pltpu.make_async_copy(
                x_hbm.at[pl.ds(base, _CHUNK)], bufs[b], sin[b]).wait()
            pltpu.async_copy(bufs[b], xo_hbm.at[pl.ds(base + c * _CHUNK,
                                                      _CHUNK)], sout[b])
            mx = compute(bufs[b], mx)
            pltpu.make_async_copy(
                x_hbm.at[pl.ds(base, _CHUNK)], bufs[b], sout[b]).wait()
            start_in(c + 4, bufs[b], sin[b])
        return mx

    mx = lax.fori_loop(0, nchunk // 4, outer, jnp.zeros((_L,), jnp.int32))
    for b in range(4):
        pltpu.make_async_copy(
            x_hbm.at[pl.ds(base, _CHUNK)], bufs[b], sin[b]).wait()
    max_v[...] = mx
    pltpu.sync_copy(hist_v, hist_hbm.at[_wid()])
    pltpu.sync_copy(max_v, max_hbm.at[_wid()])


def _masked_pass_body(n, shift, x_hbm, b_hbm, hist_hbm,
                      b0, b1, hist_all, b_v, si0, si1):
    per = n // _NW
    base = _wid() * per
    nchunk = per // _CHUNK
    bufs = (b0, b1)
    sin = (si0, si1)
    hists = [hist_all.at[pl.ds(q * _H23, _H23)] for q in range(3)]
    _zero_hist(hist_all, 3 * _H23)
    pltpu.sync_copy(b_hbm, b_v)
    lane = lax.iota(jnp.int32, _L)
    ones = jnp.ones((_L,), jnp.int32)
    bq = [b_v[pl.ds(q * _L, _L)] for q in range(3)]
    prefix_shift = 22 if shift == 7 else 11

    def start_in(ci, buf, sem):
        cc = jnp.minimum(ci, nchunk - 1)
        pltpu.async_copy(x_hbm.at[pl.ds(base + cc * _CHUNK, _CHUNK)], buf, sem)

    def compute(buf):
        def body(j):
            u = plsc.bitcast(buf[pl.ds(j * _L, _L)], jnp.int32)
            a = u & jnp.int32(0x7FFFFFFF)
            pre = a >> prefix_shift
            if shift >= 0:
                f = a >> shift
            else:
                f = a << -shift
            idx = (f & jnp.int32(0x7FF0)) | lane
            for q in range(3):
                plsc.addupdate_scatter(hists[q], [idx], ones,
                                       mask=pre == bq[q])
        plsc.parallel_loop(0, _CHUNK // _L, unroll=16)(body)

    for b in range(2):
        start_in(jnp.int32(b), bufs[b], sin[b])

    def outer(i2, carry):
        c0 = 2 * i2
        for b in range(2):
            pltpu.make_async_copy(
                x_hbm.at[pl.ds(base, _CHUNK)], bufs[b], sin[b]).wait()
            compute(bufs[b])
            start_in(c0 + b + 2, bufs[b], sin[b])
        return carry

    lax.fori_loop(0, nchunk // 2, outer, 0)
    for b in range(2):
        pltpu.make_async_copy(
            x_hbm.at[pl.ds(base, _CHUNK)], bufs[b], sin[b]).wait()
    pltpu.sync_copy(hist_all, hist_hbm.at[_wid()])


def _search(h, widx, nbuckets, steps, k, wpb):
    def cnt(m):
        return jnp.sum(jnp.where(widx < m * wpb, h, 0))
    def body(_, lohi):
        lo, hi = lohi
        mid = (lo + hi) // 2
        le = cnt(mid) <= k
        return jnp.where(le, mid, lo), jnp.where(le, hi, mid)
    lo, _ = lax.fori_loop(0, steps, body,
                          (jnp.int32(0), jnp.int32(nbuckets)))
    return lo, k - cnt(lo)


def _widx(r, c):
    return (lax.broadcasted_iota(jnp.int32, (r, c), 0) * c
            + lax.broadcasted_iota(jnp.int32, (r, c), 1))


def _sel_rows(vals):
    row = lax.broadcasted_iota(jnp.int32, (8, 128), 0)
    sel = jnp.zeros((8, 128), jnp.int32)
    for i, v in enumerate(vals):
        sel = jnp.where(row == i, v, sel)
    return sel


def _s1_body(ks, hist_ref, maxes_ref, maxbuf_ref, sel_ref, newmax_ref):
    h = jnp.sum(hist_ref[...], axis=0)
    widx = _widx(512, 128)
    vals = [None] * 6
    for q, k in enumerate(ks):
        b, r = _search(h, widx, _B1, 9, jnp.int32(k), _U1 * _L)
        vals[q], vals[3 + q] = b, r
    sel_ref[...] = _sel_rows(vals)
    mx = jnp.max(maxes_ref[...])
    mxf = lax.bitcast_convert_type(jnp.full((1, 1), mx, jnp.int32),
                                   jnp.float32)
    g = jnp.float32(_GAMMA)
    newmax_ref[...] = maxbuf_ref[...] * g + mxf * (1.0 - g)


def _s2_body(hist_ref, sel1_ref, sel_ref):
    h = jnp.sum(hist_ref[...], axis=0)
    widx = _widx(256, 128)
    vals = [None] * 6
    for q in range(3):
        b1 = jnp.max(sel1_ref[q:q + 1, :])
        r1 = jnp.max(sel1_ref[3 + q:4 + q, :])
        b2, r2 = _search(h[q], widx, _B2, 11, r1, _L)
        vals[q] = (b1 << 11) | b2
        vals[3 + q] = r2
    sel_ref[...] = _sel_rows(vals)


def _s3_body(hist_ref, sel2_ref, b1_ref, b2_ref, b3_ref, o1_ref, o2_ref,
             o3_ref):
    h = jnp.sum(hist_ref[...], axis=0)
    widx = _widx(256, 128)
    g = jnp.float32(_GAMMA)
    for q, (bref, oref) in enumerate(((b1_ref, o1_ref), (b2_ref, o2_ref),
                                      (b3_ref, o3_ref))):
        b22 = jnp.max(sel2_ref[q:q + 1, :])
        r2 = jnp.max(sel2_ref[3 + q:4 + q, :])
        b3, _ = _search(h[q], widx, _B3, 11, r2, _L)
        bits = jnp.full((1, 1), (b22 << 11) | b3, jnp.int32)
        val = lax.bitcast_convert_type(bits, jnp.float32)
        oref[...] = bref[...] * g + val * (1.0 - g)


@jax.jit
def _observer_stats(xf, max_buf, p99_99_buf, p99_9_buf, p99_buf):
    n = xf.size
    assert n % (_NW * 4 * _CHUNK) == 0
    mesh = plsc.VectorSubcoreMesh(**_MESH)
    sc_params = pltpu.CompilerParams(needs_layout_passes=False)
    f32, i32 = jnp.float32, jnp.int32

    p1 = pl.kernel(
        functools.partial(_p1_body, n),
        out_type=[jax.ShapeDtypeStruct((_NW, _H1), i32),
                  jax.ShapeDtypeStruct((_NW, _L), i32),
                  jax.ShapeDtypeStruct((n,), f32)],
        mesh=mesh,
        compiler_params=sc_params,
        scratch_types=[pltpu.VMEM((_CHUNK,), f32)] * 4
                      + [pltpu.VMEM((_H1,), i32), pltpu.VMEM((_L,), i32)]
                      + [pltpu.SemaphoreType.DMA] * 8)
    hist1, maxes, xo = p1(xf)

    ks = tuple(round(q * n) - 1 for q in (0.9999, 0.999, 0.99))
    sel1, new_max = pl.pallas_call(
        functools.partial(_s1_body, ks),
        out_shape=[jax.ShapeDtypeStruct((8, 128), i32),
                   jax.ShapeDtypeStruct((1, 1), f32)],
    )(hist1.reshape(_NW, 512, 128), maxes, max_buf.reshape(1, 1))

    def masked_pass(shift):
        return pl.kernel(
            functools.partial(_masked_pass_body, n, shift),
            out_type=jax.ShapeDtypeStruct((_NW, 3 * _H23), i32),
            mesh=mesh,
            compiler_params=sc_params,
            scratch_types=[pltpu.VMEM((_CHUNK,), f32)] * 2
                          + [pltpu.VMEM((3 * _H23,), i32)]
                          + [pltpu.VMEM((3 * _L,), i32)]
                          + [pltpu.SemaphoreType.DMA] * 2)

    hist2 = masked_pass(7)(xf, sel1[0:3, 0:_L].reshape(3 * _L))

    sel2 = pl.pallas_call(
        _s2_body,
        out_shape=jax.ShapeDtypeStruct((8, 128), i32),
    )(hist2.reshape(_NW, 3, 256, 128), sel1)

    hist3 = masked_pass(-4)(xf, sel2[0:3, 0:_L].reshape(3 * _L))

    o1, o2, o3 = pl.pallas_call(
        _s3_body,
        out_shape=[jax.ShapeDtypeStruct((1, 1), f32)] * 3,
    )(hist3.reshape(_NW, 3, 256, 128), sel2, p99_99_buf.reshape(1, 1),
      p99_9_buf.reshape(1, 1), p99_buf.reshape(1, 1))

    return (xo, new_max.reshape(()), o1.reshape(()), o2.reshape(()),
            o3.reshape(()))


def kernel(x, max_buf, p99_99_buf, p99_9_buf, p99_buf):
    xo, new_max, o1, o2, o3 = _observer_stats(
        x.reshape(-1), max_buf, p99_99_buf, p99_9_buf, p99_buf)
    return (xo.reshape(x.shape), new_max, o1, o2, o3)


# speedup vs baseline: 113.2376x; 1.3988x over previous
import functools

import jax
import jax.numpy as jnp
from jax import lax
from jax.experimental import pallas as pl
from jax.experimental.pallas import tpu as pltpu
from jax.experimental.pallas import tpu_sc as plsc

_GAMMA = 0.99
_NC, _NS, _L = 2, 16, 16
_NW = _NC * _NS
_CR, _CC = 8, 1024

_B1, _B2, _B3 = 512, 2048, 2048
_R23 = 3 * _B2 // 8

_MESH = dict(core_axis_name="c", subcore_axis_name="s",
             num_cores=_NC, num_subcores=_NS)


def _wid():
    return lax.axis_index("s") * _NC + lax.axis_index("c")


def _zero_hist(hist_v, nrows):
    zeros = jnp.zeros((_L,), jnp.int32)
    @plsc.parallel_loop(0, nrows * 8, unroll=8)
    def _(i):
        hist_v[i >> 3, pl.ds((i & 7) * _L, _L)] = zeros


def _p1_body(nrows, x_hbm, hist_hbm, max_hbm, xo_hbm,
             b0, b1, b2, b3, hist_v, max_v,
             si0, si1, si2, si3, so0, so1, so2, so3):
    rpw = nrows // _NW
    row_base = _wid() * rpw
    nchunk = (rpw // _CR) * (4096 // _CC)
    ncc = 4096 // _CC
    bufs = (b0, b1, b2, b3)
    sin = (si0, si1, si2, si3)
    sout = (so0, so1, so2, so3)
    _zero_hist(hist_v, _B1)
    lane = lax.iota(jnp.int32, _L)
    ones = jnp.ones((_L,), jnp.int32)

    def chunk_slice(ci):
        cc = jnp.minimum(ci, nchunk - 1)
        r0 = row_base + (cc // ncc) * _CR
        c0 = (cc % ncc) * _CC
        return (pl.ds(r0, _CR), pl.ds(c0, _CC))

    def start_in(ci, buf, sem):
        pltpu.async_copy(x_hbm.at[chunk_slice(ci)], buf, sem)

    def wait(buf, sem):
        pltpu.make_async_copy(x_hbm.at[chunk_slice(0)], buf, sem).wait()

    def compute(buf, mx):
        for r in range(_CR):
            def body(j, mx, r=r):
                u = plsc.bitcast(buf[r, pl.ds(j * _L, _L)], jnp.int32)
                a = u & jnp.int32(0x7FFFFFFF)
                mx = jnp.maximum(mx, a)
                co = ((j.astype(jnp.int32) & 7) << 4) | lane
                plsc.addupdate_scatter(hist_v, [a >> 22, co], ones)
                return mx
            mx = plsc.parallel_loop(0, _CC // _L, carry=mx, unroll=16)(body)
        return mx

    for b in range(4):
        start_in(jnp.int32(b), bufs[b], sin[b])

    def outer(i4, mx):
        c0 = 4 * i4
        for b in range(4):
            c = c0 + b
            wait(bufs[b], sin[b])
            pltpu.async_copy(bufs[b], xo_hbm.at[chunk_slice(c)], sout[b])
            mx = compute(bufs[b], mx)
            wait(bufs[b], sout[b])
            start_in(c + 4, bufs[b], sin[b])
        return mx

    mx = lax.fori_loop(0, nchunk // 4, outer, jnp.zeros((_L,), jnp.int32))
    for b in range(4):
        wait(bufs[b], sin[b])
    max_v[...] = mx
    pltpu.sync_copy(hist_v, hist_hbm.at[pl.ds(_wid() * _B1, _B1), :])
    pltpu.sync_copy(max_v, max_hbm.at[_wid()])


def _masked_pass_body(nrows, shift, x_hbm, b_hbm, hist_hbm,
                      b0, b1, hist_all, b_v, si0, si1):
    rpw = nrows // _NW
    row_base = _wid() * rpw
    nchunk = (rpw // _CR) * (4096 // _CC)
    ncc = 4096 // _CC
    bufs = (b0, b1)
    sin = (si0, si1)
    nr = _B2 // 8
    hists = [hist_all.at[pl.ds(q * nr, nr), :] for q in range(3)]
    _zero_hist(hist_all, _R23)
    pltpu.sync_copy(b_hbm, b_v)
    lane = lax.iota(jnp.int32, _L)
    ones = jnp.ones((_L,), jnp.int32)
    bq = [b_v[pl.ds(q * _L, _L)] for q in range(3)]
    prefix_shift = shift + 11

    def chunk_slice(ci):
        cc = jnp.minimum(ci, nchunk - 1)
        r0 = row_base + (cc // ncc) * _CR
        c0 = (cc % ncc) * _CC
        return (pl.ds(r0, _CR), pl.ds(c0, _CC))

    def start_in(ci, buf, sem):
        pltpu.async_copy(x_hbm.at[chunk_slice(ci)], buf, sem)

    def wait(buf, sem):
        pltpu.make_async_copy(x_hbm.at[chunk_slice(0)], buf, sem).wait()

    def compute(buf):
        for r in range(_CR):
            def body(j, r=r):
                u = plsc.bitcast(buf[r, pl.ds(j * _L, _L)], jnp.int32)
                a = u & jnp.int32(0x7FFFFFFF)
                pre = a >> prefix_shift
                i0 = (a >> (shift + 3)) & jnp.int32(0xFF)
                i1 = (((a >> shift) & jnp.int32(7)) << 4) | lane
                for q in range(3):
                    plsc.addupdate_scatter(hists[q], [i0, i1], ones,
                                           mask=pre == bq[q])
            plsc.parallel_loop(0, _CC // _L, unroll=16)(body)

    for b in range(2):
        start_in(jnp.int32(b), bufs[b], sin[b])

    def outer(i2, carry):
        c0 = 2 * i2
        for b in range(2):
            wait(bufs[b], sin[b])
            compute(bufs[b])
            start_in(c0 + b + 2, bufs[b], sin[b])
        return carry

    lax.fori_loop(0, nchunk // 2, outer, 0)
    for b in range(2):
        wait(bufs[b], sin[b])
    pltpu.sync_copy(hist_all, hist_hbm.at[pl.ds(_wid() * _R23, _R23), :])


def _search(h, widx, nbuckets, steps, k, wpb):
    def cnt(m):
        return jnp.sum(jnp.where(widx < m * wpb, h, 0))
    def body(_, lohi):
        lo, hi = lohi
        mid = (lo + hi) // 2
        le = cnt(mid) <= k
        return jnp.where(le, mid, lo), jnp.where(le, hi, mid)
    lo, _ = lax.fori_loop(0, steps, body,
                          (jnp.int32(0), jnp.int32(nbuckets)))
    return lo, k - cnt(lo)


def _widx(r, c):
    return (lax.broadcasted_iota(jnp.int32, (r, c), 0) * c
            + lax.broadcasted_iota(jnp.int32, (r, c), 1))


def _sel_rows(vals):
    row = lax.broadcasted_iota(jnp.int32, (8, 128), 0)
    sel = jnp.zeros((8, 128), jnp.int32)
    for i, v in enumerate(vals):
        sel = jnp.where(row == i, v, sel)
    return sel


def _s1_body(ks, hist_ref, maxes_ref, maxbuf_ref, sel_ref, newmax_ref):
    h = jnp.sum(hist_ref[...], axis=0)
    widx = _widx(512, 128)
    vals = [None] * 6
    for q, k in enumerate(ks):
        b, r = _search(h, widx, _B1, 9, jnp.int32(k), 128)
        vals[q], vals[3 + q] = b, r
    sel_ref[...] = _sel_rows(vals)
    mx = jnp.max(maxes_ref[...])
    mxf = lax.bitcast_convert_type(jnp.full((1, 1), mx, jnp.int32),
                                   jnp.float32)
    g = jnp.float32(_GAMMA)
    newmax_ref[...] = maxbuf_ref[...] * g + mxf * (1.0 - g)


def _s2_body(hist_ref, sel1_ref, sel_ref):
    h = jnp.sum(hist_ref[...], axis=0)
    widx = _widx(256, 128)
    vals = [None] * 6
    for q in range(3):
        b1 = jnp.max(sel1_ref[q:q + 1, :])
        r1 = jnp.max(sel1_ref[3 + q:4 + q, :])
        b2, r2 = _search(h[q], widx, _B2, 11, r1, _L)
        vals[q] = (b1 << 11) | b2
        vals[3 + q] = r2
    sel_ref[...] = _sel_rows(vals)


def _s3_body(hist_ref, sel2_ref, b1_ref, b2_ref, b3_ref, o1_ref, o2_ref,
             o3_ref):
    h = jnp.sum(hist_ref[...], axis=0)
    widx = _widx(256, 128)
    g = jnp.float32(_GAMMA)
    for q, (bref, oref) in enumerate(((b1_ref, o1_ref), (b2_ref, o2_ref),
                                      (b3_ref, o3_ref))):
        b22 = jnp.max(sel2_ref[q:q + 1, :])
        r2 = jnp.max(sel2_ref[3 + q:4 + q, :])
        b3, _ = _search(h[q], widx, _B3, 11, r2, _L)
        bits = jnp.full((1, 1), (b22 << 11) | b3, jnp.int32)
        val = lax.bitcast_convert_type(bits, jnp.float32)
        oref[...] = bref[...] * g + val * (1.0 - g)


@jax.jit
def _observer_stats(x2, max_buf, p99_99_buf, p99_9_buf, p99_buf):
    nrows = x2.shape[0]
    n = x2.size
    assert x2.shape[1] == 4096 and nrows % (_NW * _CR) == 0
    mesh = plsc.VectorSubcoreMesh(**_MESH)
    sc_params = pltpu.CompilerParams(needs_layout_passes=False,
                                     use_tc_tiling_on_sc=True)
    f32, i32 = jnp.float32, jnp.int32

    p1 = pl.kernel(
        functools.partial(_p1_body, nrows),
        out_type=[jax.ShapeDtypeStruct((_NW * _B1, 128), i32),
                  jax.ShapeDtypeStruct((_NW, _L), i32),
                  jax.ShapeDtypeStruct(x2.shape, f32)],
        mesh=mesh,
        compiler_params=sc_params,
        scratch_types=[pltpu.VMEM((_CR, _CC), f32)] * 4
                      + [pltpu.VMEM((_B1, 128), i32), pltpu.VMEM((_L,), i32)]
                      + [pltpu.SemaphoreType.DMA] * 8)
    hist1, maxes, xo = p1(x2)

    ks = tuple(round(q * n) - 1 for q in (0.9999, 0.999, 0.99))
    sel1, new_max = pl.pallas_call(
        functools.partial(_s1_body, ks),
        out_shape=[jax.ShapeDtypeStruct((8, 128), i32),
                   jax.ShapeDtypeStruct((1, 1), f32)],
    )(hist1.reshape(_NW, _B1, 128), maxes, max_buf.reshape(1, 1))

    def masked_pass(shift):
        return pl.kernel(
            functools.partial(_masked_pass_body, nrows, shift),
            out_type=jax.ShapeDtypeStruct((_NW * _R23, 128), i32),
            mesh=mesh,
            compiler_params=sc_params,
            scratch_types=[pltpu.VMEM((_CR, _CC), f32)] * 2
                          + [pltpu.VMEM((_R23, 128), i32)]
                          + [pltpu.VMEM((3 * _L,), i32)]
                          + [pltpu.SemaphoreType.DMA] * 2)

    hist2 = masked_pass(11)(x2, sel1[0:3, 0:_L].reshape(3 * _L))

    sel2 = pl.pallas_call(
        _s2_body,
        out_shape=jax.ShapeDtypeStruct((8, 128), i32),
    )(hist2.reshape(_NW, 3, 256, 128), sel1)

    hist3 = masked_pass(0)(x2, sel2[0:3, 0:_L].reshape(3 * _L))

    o1, o2, o3 = pl.pallas_call(
        _s3_body,
        out_shape=[jax.ShapeDtypeStruct((1, 1), f32)] * 3,
    )(hist3.reshape(_NW, 3, 256, 128), sel2, p99_99_buf.reshape(1, 1),
      p99_9_buf.reshape(1, 1), p99_buf.reshape(1, 1))

    return (xo, new_max.reshape(()), o1.reshape(()), o2.reshape(()),
            o3.reshape(()))


def kernel(x, max_buf, p99_99_buf, p99_9_buf, p99_buf):
    x2 = x.reshape(-1, 4096)
    xo, new_max, o1, o2, o3 = _observer_stats(
        x2, max_buf, p99_99_buf, p99_9_buf, p99_buf)
    return (xo.reshape(x.shape), new_max, o1, o2, o3)
